# baseline (speedup 1.0000x reference)
import functools

import jax
import jax.numpy as jnp
from jax.experimental import pallas as pl

N_ATOM, N_RES, N_POS = 55, 21, 24
RES_OFF, POS_OFF = 64, 96
B1_ROW = 120
C = 128
H = 3 * C
TOK_BLOCK = 4096


def _prep_body(atom_ref, res_ref, pos_ref, w1_ref, b1_ref, w2_ref, b2_ref,
               combt_ref, w2ht_ref):
    ca = jnp.dot(atom_ref[:], w1_ref[0:C, :],
                 preferred_element_type=jnp.float32)
    cr = jnp.dot(res_ref[:], w1_ref[C:2 * C, :],
                 preferred_element_type=jnp.float32)
    cp = jnp.dot(pos_ref[:], w1_ref[2 * C:3 * C, :],
                 preferred_element_type=jnp.float32)
    z = lambda k: jnp.zeros((k, H), dtype=jnp.float32)
    pieces = [ca, z(RES_OFF - N_ATOM), cr, z(POS_OFF - RES_OFF - N_RES), cp,
              z(B1_ROW - POS_OFF - N_POS), b1_ref[:], z(C - B1_ROW - 1)]
    comb = jnp.concatenate([p for p in pieces if p.shape[0] > 0], axis=0)
    combt_ref[:] = comb.T.astype(jnp.bfloat16)
    w2ht = jnp.concatenate(
        [(w2_ref[:] * 0.5).T, b2_ref[:].T, jnp.zeros((C, 7), jnp.float32)],
        axis=1)
    w2ht_ref[:] = w2ht.astype(jnp.bfloat16)


def _main_body(atom_ref, res_ref, pos_ref, combt_ref, w2ht_ref, out_ref):
    _, tp, _ = out_ref.shape
    va = atom_ref[0]
    vr = res_ref[0] + RES_OFF
    vp = pos_ref[0] + POS_OFF
    iota = jax.lax.broadcasted_iota(jnp.int32, (C, tp), 0)
    hit = (iota == va) | (iota == vr) | (iota == vp) | (iota == B1_ROW)
    oht = hit.astype(jnp.bfloat16)
    ht = jnp.dot(combt_ref[:], oht, preferred_element_type=jnp.float32)
    gt = ht * (1.0 + jax.lax.erf(ht * 0.7071067811865476))
    gt2 = jnp.concatenate(
        [gt.astype(jnp.bfloat16),
         jnp.ones((8, tp), dtype=jnp.bfloat16)], axis=0)
    g2t = jnp.dot(w2ht_ref[:], gt2, preferred_element_type=jnp.float32)
    out_ref[0] = g2t.T


@functools.partial(jax.jit, static_argnames=())
def kernel(atom_type, aa_type, aa_pos, atom_table, residue_table, pos_table,
           W1, b1, W2, b2):
    b, l = atom_type.shape
    tp = TOK_BLOCK
    grid = (l, b // tp)

    combt, w2ht = pl.pallas_call(
        _prep_body,
        out_shape=(jax.ShapeDtypeStruct((H, C), jnp.bfloat16),
                   jax.ShapeDtypeStruct((C, H + 8), jnp.bfloat16)),
    )(atom_table, residue_table, pos_table, W1, b1.reshape(1, H), W2,
      b2.reshape(1, C))

    idx_spec = pl.BlockSpec((1, 1, tp), lambda j, i: (j, 0, i))
    full = lambda shape: pl.BlockSpec(shape, lambda j, i: (0, 0))

    out = pl.pallas_call(
        _main_body,
        grid=grid,
        in_specs=[
            idx_spec, idx_spec, idx_spec,
            full((H, C)),
            full((C, H + 8)),
        ],
        out_specs=pl.BlockSpec((1, tp, C), lambda j, i: (j, i, 0)),
        out_shape=jax.ShapeDtypeStruct((l, b, C), jnp.float32),
    )(atom_type.T.reshape(l, 1, b), aa_type.T.reshape(l, 1, b),
      aa_pos.T.reshape(l, 1, b), combt, w2ht)

    return out.transpose(1, 0, 2)
